# baseline (speedup 1.0000x reference)
import functools

import jax
import jax.experimental.layout as jlayout
import jax.numpy as jnp
from jax import lax
from jax.experimental import pallas as pl
from jax.experimental.pallas import tpu as pltpu
from jax.experimental.pallas import tpu_sc as plsc

NC, NS, L = 2, 16, 16
NW = NC * NS
B, H, D = 4096, 200, 64
D2 = 2 * D
RPW = B // NW
EPW = RPW * H
CH = 64
NCHUNK = EPW // CH
NBUF = 5
LA = NBUF - 2
ACC_ROWS = NS * RPW
TRASH0 = 2 * ACC_ROWS
OB = 8


def _iota16():
    return lax.broadcasted_iota(jnp.int32, (L,), 0)


def _body(ids_hbm, mask_hbm, table_hbm, out_hbm,
          ids_v, mask_v, inv_v, blk_e, blk_o, acc_sh, *ring):
    gbufs = ring[:NBUF]
    dsts = ring[NBUF:2 * NBUF]
    sem_g = ring[2 * NBUF:3 * NBUF]
    sem_s = ring[3 * NBUF:4 * NBUF]

    c = lax.axis_index("c")
    s = lax.axis_index("s")
    wid = c * NS + s
    ebase = wid * EPW
    row_base = wid * RPW
    slot_base = s * RPW
    trash = TRASH0 + s

    pltpu.sync_copy(ids_hbm.at[pl.ds(ebase, EPW)], ids_v)
    pltpu.sync_copy(mask_hbm.at[pl.ds(ebase, EPW)], mask_v.at[pl.ds(0, EPW)])

    def pre_body(t, _):
        idv = ids_v[pl.ds(t * L, L)]
        m = mask_v[pl.ds(t * L, L)]
        ids_v[pl.ds(t * L, L)] = lax.shift_right_logical(idv, 1)
        mask_v[pl.ds(t * L, L)] = m | lax.shift_left(idv & 1, 1)
        return 0
    lax.fori_loop(0, EPW // L, pre_body, 0)

    zeros = jnp.zeros((L,), jnp.float32)
    for i in range(OB):
        for j in range(D2 // L):
            blk_e[i, pl.ds(j * L, L)] = zeros

    def zero_body(g, _):
        o = pl.multiple_of(slot_base + g * OB, OB)
        pltpu.sync_copy(blk_e, acc_sh.at[pl.ds(o, OB)])
        pltpu.sync_copy(
            blk_e, acc_sh.at[pl.ds(pl.multiple_of(ACC_ROWS + o, OB), OB)])
        return 0
    lax.fori_loop(0, RPW // OB, zero_body, 0)

    def inv_body(r, _):
        cntv = jnp.zeros((L,), jnp.int32)
        for k in range(H // L):
            mb = (mask_v[pl.ds(r * H + k * L, L)] & 1) > 0
            cntv = cntv + plsc.all_reduce_population_count(mb)
        tail = (mask_v[pl.ds(r * H + (H // L) * L, L)] & 1) > 0
        tail = jnp.logical_and(tail, _iota16() < (H % L))
        cntv = cntv + plsc.all_reduce_population_count(tail)
        cf = jnp.maximum(cntv.astype(jnp.float32), 1.0)
        inv_v[pl.ds(r * L, L)] = 1.0 / cf
        return 0
    lax.fori_loop(0, RPW, inv_body, 0)

    def gather_desc(g, b):
        return pltpu.make_async_copy(
            table_hbm.at[ids_v.at[pl.ds(g * CH, CH)]], gbufs[b], sem_g[b])

    def scatter_start(b):
        pltpu.async_copy(gbufs[b], acc_sh.at[dsts[b]], sem_s[b], add=True)

    def scatter_wait(b):
        pltpu.make_async_copy(gbufs[b], acc_sh.at[dsts[b]], sem_s[b]).wait()

    for g0 in range(LA):
        gather_desc(g0, g0).start()

    def ring_body(i, _):
        for b in range(NBUF):
            g = i * NBUF + b
            b2 = (b - 2) % NBUF

            @pl.when(g >= 2)
            def _():
                scatter_wait(b2)

            @pl.when(g + LA < NCHUNK)
            def _():
                gather_desc(g + LA, b2).start()

            off = g * CH
            db = dsts[b]
            for j in range(CH // L):
                code = mask_v[pl.ds(off + j * L, L)]
                e = off + j * L + _iota16()
                slot = slot_base + e // H + lax.shift_left(code >> 1, 11)
                db[pl.ds(j * L, L)] = jnp.where((code & 1) > 0, slot, trash)
            gather_desc(g, b).wait()
            scatter_start(b)
        return 0
    lax.fori_loop(0, NCHUNK // NBUF, ring_body, 0)

    scatter_wait((NCHUNK - 2) % NBUF)
    scatter_wait((NCHUNK - 1) % NBUF)

    def out_body(gb, _):
        o = pl.multiple_of(slot_base + gb * OB, OB)
        pltpu.sync_copy(acc_sh.at[pl.ds(o, OB)], blk_e)
        pltpu.sync_copy(
            acc_sh.at[pl.ds(pl.multiple_of(ACC_ROWS + o, OB), OB)], blk_o)
        for i in range(OB):
            inv = inv_v[pl.ds(gb * (OB * L) + i * L, L)]
            for j in range(D // L):
                v = blk_e[i, pl.ds(j * L, L)] + blk_o[i, pl.ds(D + j * L, L)]
                blk_e[i // 2, pl.ds((i % 2) * D + j * L, L)] = v * inv
        pltpu.sync_copy(
            blk_e.at[pl.ds(0, OB // 2)],
            out_hbm.at[pl.ds(
                pl.multiple_of((row_base + gb * OB) // 2, OB // 2), OB // 2)])
        return 0
    lax.fori_loop(0, RPW // OB, out_body, 0)


@jax.jit
def _sc_pool(ids_flat, mask_flat, table2):
    mesh = plsc.VectorSubcoreMesh(core_axis_name="c", subcore_axis_name="s")
    f = pl.kernel(
        _body,
        out_type=jax.ShapeDtypeStruct((B // 2, D2), jnp.float32),
        mesh=mesh,
        compiler_params=pltpu.CompilerParams(needs_layout_passes=False,
                                             use_tc_tiling_on_sc=True),
        scratch_types=(
            [
                pltpu.VMEM((EPW,), jnp.int32),
                pltpu.VMEM((EPW + L,), jnp.int32),
                pltpu.VMEM((RPW * L,), jnp.float32),
                pltpu.VMEM((OB, D2), jnp.float32),
                pltpu.VMEM((OB, D2), jnp.float32),
                pltpu.VMEM_SHARED((2 * ACC_ROWS + NS, D2),
                                  jnp.float32),
            ]
            + [pltpu.VMEM((CH, D2), jnp.float32)] * NBUF
            + [pltpu.VMEM((CH,), jnp.int32)] * NBUF
            + [pltpu.SemaphoreType.DMA] * (2 * NBUF)
        ),
    )
    return f(ids_flat, mask_flat, table2)


def kernel(input_ids, attention_mask, table):
    ids_flat = input_ids.reshape(-1)
    mask_flat = attention_mask.reshape(-1)
    table2 = table.reshape(table.shape[0] // 2, D2)
    table2 = jlayout.with_layout_constraint(
        table2, jlayout.Layout(major_to_minor=(0, 1)))
    return _sc_pool(ids_flat, mask_flat, table2).reshape(B, D)


# speedup vs baseline: 1.0124x; 1.0124x over previous
import functools

import jax
import jax.experimental.layout as jlayout
import jax.numpy as jnp
from jax import lax
from jax.experimental import pallas as pl
from jax.experimental.pallas import tpu as pltpu
from jax.experimental.pallas import tpu_sc as plsc

NC, NS, L = 2, 16, 16
NW = NC * NS
B, H, D = 4096, 200, 64
D2 = 2 * D
RPW = B // NW
EPW = RPW * H
CH = 64
NCHUNK = EPW // CH
NBUF = 8
LA = NBUF - 2
ACC_ROWS = NS * RPW
TRASH0 = 2 * ACC_ROWS
OB = 8


def _iota16():
    return lax.broadcasted_iota(jnp.int32, (L,), 0)


def _body(ids_hbm, mask_hbm, table_hbm, out_hbm,
          mask_v, inv_v, blk_e, blk_o, acc_sh, *ring):
    gbufs = ring[:NBUF]
    idxs = ring[NBUF:2 * NBUF]
    dsts = ring[2 * NBUF:3 * NBUF]
    sem_g = ring[3 * NBUF:4 * NBUF]
    sem_s = ring[4 * NBUF:5 * NBUF]
    sem_i = ring[5 * NBUF:6 * NBUF]

    c = lax.axis_index("c")
    s = lax.axis_index("s")
    wid = c * NS + s
    ebase = wid * EPW
    row_base = wid * RPW
    slot_base = s * RPW
    trash = TRASH0 + s

    pltpu.sync_copy(mask_hbm.at[pl.ds(ebase, EPW)], mask_v.at[pl.ds(0, EPW)])

    zeros = jnp.zeros((L,), jnp.float32)
    for i in range(OB):
        for j in range(D2 // L):
            blk_e[i, pl.ds(j * L, L)] = zeros

    def zero_body(g, _):
        o = pl.multiple_of(slot_base + g * OB, OB)
        pltpu.sync_copy(blk_e, acc_sh.at[pl.ds(o, OB)])
        pltpu.sync_copy(
            blk_e, acc_sh.at[pl.ds(pl.multiple_of(ACC_ROWS + o, OB), OB)])
        return 0
    lax.fori_loop(0, RPW // OB, zero_body, 0)

    def inv_body(r, _):
        cntv = jnp.zeros((L,), jnp.int32)
        for k in range(H // L):
            mb = mask_v[pl.ds(r * H + k * L, L)] > 0
            cntv = cntv + plsc.all_reduce_population_count(mb)
        tail = mask_v[pl.ds(r * H + (H // L) * L, L)] > 0
        tail = jnp.logical_and(tail, _iota16() < (H % L))
        cntv = cntv + plsc.all_reduce_population_count(tail)
        cf = jnp.maximum(cntv.astype(jnp.float32), 1.0)
        inv_v[pl.ds(r * L, L)] = 1.0 / cf
        return 0
    lax.fori_loop(0, RPW, inv_body, 0)

    def stage_start(g, b):
        pltpu.async_copy(
            ids_hbm.at[pl.ds(ebase + g * CH, CH)], idxs[b], sem_i[b])

    def stage_wait(g, b):
        pltpu.make_async_copy(
            ids_hbm.at[pl.ds(ebase + g * CH, CH)], idxs[b], sem_i[b]).wait()

    def prep_and_fire(g, b):
        off = g * CH
        db = dsts[b]
        for j in range(CH // L):
            idv = idxs[b][pl.ds(j * L, L)]
            m = mask_v[pl.ds(off + j * L, L)]
            e = off + j * L + _iota16()
            slot = slot_base + e // H + lax.shift_left(idv & 1, 11)
            db[pl.ds(j * L, L)] = jnp.where(m > 0, slot, trash)
            idxs[b][pl.ds(j * L, L)] = lax.shift_right_logical(idv, 1)
        pltpu.make_async_copy(
            table_hbm.at[idxs[b]], gbufs[b], sem_g[b]).start()

    def gather_wait(b):
        pltpu.make_async_copy(
            table_hbm.at[idxs[b]], gbufs[b], sem_g[b]).wait()

    def scatter_start(b):
        pltpu.async_copy(gbufs[b], acc_sh.at[dsts[b]], sem_s[b], add=True)

    def scatter_wait(b):
        pltpu.make_async_copy(gbufs[b], acc_sh.at[dsts[b]], sem_s[b]).wait()

    for g0 in range(NBUF):
        stage_start(g0, g0)
    for g0 in range(LA):
        stage_wait(g0, g0)
        prep_and_fire(g0, g0)

    def ring_body(i, _):
        for b in range(NBUF):
            g = i * NBUF + b
            b2 = (b - 2) % NBUF

            @pl.when(g >= 2)
            def _():
                scatter_wait(b2)

            @pl.when(g + LA < NCHUNK)
            def _():
                stage_wait(g + LA, b2)
                prep_and_fire(g + LA, b2)

            gather_wait(b)
            scatter_start(b)

            @pl.when(g + NBUF < NCHUNK)
            def _():
                stage_start(g + NBUF, b)
        return 0
    lax.fori_loop(0, NCHUNK // NBUF, ring_body, 0)

    scatter_wait((NCHUNK - 2) % NBUF)
    scatter_wait((NCHUNK - 1) % NBUF)

    def out_body(gb, _):
        o = pl.multiple_of(slot_base + gb * OB, OB)
        pltpu.sync_copy(acc_sh.at[pl.ds(o, OB)], blk_e)
        pltpu.sync_copy(
            acc_sh.at[pl.ds(pl.multiple_of(ACC_ROWS + o, OB), OB)], blk_o)
        for i in range(OB):
            inv = inv_v[pl.ds(gb * (OB * L) + i * L, L)]
            for j in range(D // L):
                v = blk_e[i, pl.ds(j * L, L)] + blk_o[i, pl.ds(D + j * L, L)]
                blk_e[i // 2, pl.ds((i % 2) * D + j * L, L)] = v * inv
        pltpu.sync_copy(
            blk_e.at[pl.ds(0, OB // 2)],
            out_hbm.at[pl.ds(
                pl.multiple_of((row_base + gb * OB) // 2, OB // 2), OB // 2)])
        return 0
    lax.fori_loop(0, RPW // OB, out_body, 0)


@jax.jit
def _sc_pool(ids_flat, mask_flat, table2):
    mesh = plsc.VectorSubcoreMesh(core_axis_name="c", subcore_axis_name="s")
    f = pl.kernel(
        _body,
        out_type=jax.ShapeDtypeStruct((B // 2, D2), jnp.float32),
        mesh=mesh,
        compiler_params=pltpu.CompilerParams(needs_layout_passes=False,
                                             use_tc_tiling_on_sc=True),
        scratch_types=(
            [
                pltpu.VMEM((EPW + L,), jnp.int32),
                pltpu.VMEM((RPW * L,), jnp.float32),
                pltpu.VMEM((OB, D2), jnp.float32),
                pltpu.VMEM((OB, D2), jnp.float32),
                pltpu.VMEM_SHARED((2 * ACC_ROWS + NS, D2),
                                  jnp.float32),
            ]
            + [pltpu.VMEM((CH, D2), jnp.float32)] * NBUF
            + [pltpu.VMEM((CH,), jnp.int32)] * NBUF
            + [pltpu.VMEM((CH,), jnp.int32)] * NBUF
            + [pltpu.SemaphoreType.DMA] * (3 * NBUF)
        ),
    )
    return f(ids_flat, mask_flat, table2)


def kernel(input_ids, attention_mask, table):
    ids_flat = input_ids.reshape(-1)
    mask_flat = attention_mask.reshape(-1)
    table2 = table.reshape(table.shape[0] // 2, D2)
    table2 = jlayout.with_layout_constraint(
        table2, jlayout.Layout(major_to_minor=(0, 1)))
    return _sc_pool(ids_flat, mask_flat, table2).reshape(B, D)


# speedup vs baseline: 1.1623x; 1.1481x over previous
import functools

import jax
import jax.numpy as jnp
from jax import lax
from jax.experimental import pallas as pl
from jax.experimental.pallas import tpu as pltpu
from jax.experimental.pallas import tpu_sc as plsc

NC, NS, L = 2, 16, 16
NW = NC * NS
B, H, D = 4096, 200, 64
RPW = B // NW
EPW = RPW * H
CH = 128
NCHUNK = EPW // CH
NBUF = 8
LA = NBUF - 2
ACC_ROWS = NS * RPW
TRASH0 = ACC_ROWS


def _iota16():
    return lax.broadcasted_iota(jnp.int32, (L,), 0)


def _body(ids_hbm, mask_hbm, table_hbm, out_hbm,
          ids_v, mask_v, inv_v, blk_v, acc_sh, *ring):
    gbufs = ring[:NBUF]
    dsts = ring[NBUF:2 * NBUF]
    sem_g = ring[2 * NBUF:3 * NBUF]
    sem_s = ring[3 * NBUF:4 * NBUF]

    c = lax.axis_index("c")
    s = lax.axis_index("s")
    wid = c * NS + s
    ebase = wid * EPW
    row_base = wid * RPW
    slot_base = s * RPW
    trash = TRASH0 + s

    pltpu.sync_copy(ids_hbm.at[pl.ds(ebase, EPW)], ids_v)
    pltpu.sync_copy(mask_hbm.at[pl.ds(ebase, EPW)], mask_v.at[pl.ds(0, EPW)])

    zeros = jnp.zeros((L,), jnp.float32)
    for i in range(L):
        for j in range(D // L):
            blk_v[i, pl.ds(j * L, L)] = zeros

    def zero_body(g, _):
        pltpu.sync_copy(blk_v, acc_sh.at[pl.ds(slot_base + g * L, L)])
        return 0
    lax.fori_loop(0, RPW // L, zero_body, 0)

    def inv_body(r, _):
        cntv = jnp.zeros((L,), jnp.int32)
        for k in range(H // L):
            mb = mask_v[pl.ds(r * H + k * L, L)] > 0
            cntv = cntv + plsc.all_reduce_population_count(mb)
        tail = mask_v[pl.ds(r * H + (H // L) * L, L)] > 0
        tail = jnp.logical_and(tail, _iota16() < (H % L))
        cntv = cntv + plsc.all_reduce_population_count(tail)
        cf = jnp.maximum(cntv.astype(jnp.float32), 1.0)
        inv_v[pl.ds(r * L, L)] = 1.0 / cf
        return 0
    lax.fori_loop(0, RPW, inv_body, 0)

    def gather_desc(g, b):
        return pltpu.make_async_copy(
            table_hbm.at[ids_v.at[pl.ds(g * CH, CH)]], gbufs[b], sem_g[b])

    def scatter_start(b):
        pltpu.async_copy(gbufs[b], acc_sh.at[dsts[b]], sem_s[b], add=True)

    def scatter_wait(b):
        pltpu.make_async_copy(gbufs[b], acc_sh.at[dsts[b]], sem_s[b]).wait()

    for g0 in range(LA):
        gather_desc(g0, g0).start()

    def ring_body(i, _):
        for b in range(NBUF):
            g = i * NBUF + b
            b2 = (b - 2) % NBUF

            @pl.when(g >= 2)
            def _():
                scatter_wait(b2)

            @pl.when(g + LA < NCHUNK)
            def _():
                gather_desc(g + LA, b2).start()

            off = g * CH
            db = dsts[b]
            for j in range(CH // L):
                m = mask_v[pl.ds(off + j * L, L)]
                e = off + j * L + _iota16()
                slot = slot_base + e // H
                db[pl.ds(j * L, L)] = jnp.where(m > 0, slot, trash)
            gather_desc(g, b).wait()
            scatter_start(b)
        return 0
    lax.fori_loop(0, NCHUNK // NBUF, ring_body, 0)

    scatter_wait((NCHUNK - 2) % NBUF)
    scatter_wait((NCHUNK - 1) % NBUF)

    def out_body(gb, _):
        pltpu.sync_copy(acc_sh.at[pl.ds(slot_base + gb * L, L)], blk_v)
        for i in range(L):
            inv = inv_v[pl.ds(gb * (L * L) + i * L, L)]
            for j in range(D // L):
                blk_v[i, pl.ds(j * L, L)] = blk_v[i, pl.ds(j * L, L)] * inv
        pltpu.sync_copy(blk_v, out_hbm.at[pl.ds(row_base + gb * L, L)])
        return 0
    lax.fori_loop(0, RPW // L, out_body, 0)


@jax.jit
def _sc_pool(ids_flat, mask_flat, table):
    mesh = plsc.VectorSubcoreMesh(core_axis_name="c", subcore_axis_name="s")
    f = pl.kernel(
        _body,
        out_type=jax.ShapeDtypeStruct((B, D), jnp.float32),
        mesh=mesh,
        compiler_params=pltpu.CompilerParams(needs_layout_passes=False,
                                             use_tc_tiling_on_sc=False),
        scratch_types=(
            [
                pltpu.VMEM((EPW,), jnp.int32),
                pltpu.VMEM((EPW + L,), jnp.int32),
                pltpu.VMEM((RPW * L,), jnp.float32),
                pltpu.VMEM((L, D), jnp.float32),
                pltpu.VMEM_SHARED((ACC_ROWS + NS, D), jnp.float32),
            ]
            + [pltpu.VMEM((CH, D), jnp.float32)] * NBUF
            + [pltpu.VMEM((CH,), jnp.int32)] * NBUF
            + [pltpu.SemaphoreType.DMA] * (2 * NBUF)
        ),
    )
    return f(ids_flat, mask_flat, table)


def kernel(input_ids, attention_mask, table):
    ids_flat = input_ids.reshape(-1)
    mask_flat = attention_mask.reshape(-1)
    return _sc_pool(ids_flat, mask_flat, table)
